# baseline (speedup 1.0000x reference)
import functools

import jax
import jax.numpy as jnp
from jax import lax
from jax.experimental import pallas as pl
from jax.experimental.pallas import tpu as pltpu
from jax.experimental.pallas import tpu_sc as plsc

_NC = 1
_NS = 16
_NW = _NC * _NS
_L = 16


@functools.lru_cache(maxsize=None)
def _build(n_ext: int, n_pad: int, n_out: int):
    per_w = n_out // _NW
    n_vec = per_w // _L
    steps = []
    s = 1
    while s * 2 < n_ext:
        s *= 2
    while s >= 1:
        steps.append(s)
        s //= 2

    mesh = plsc.VectorSubcoreMesh(
        core_axis_name="c", subcore_axis_name="s",
        num_cores=_NC, num_subcores=_NS,
    )

    @functools.partial(
        pl.kernel,
        out_type=jax.ShapeDtypeStruct((n_out,), jnp.float32),
        mesh=mesh,
        compiler_params=pltpu.CompilerParams(needs_layout_passes=False),
        scratch_types=[
            pltpu.VMEM((2 * n_pad,), jnp.float32),
            pltpu.VMEM((2 * per_w,), jnp.float32),
            pltpu.VMEM((per_w,), jnp.float32),
            pltpu.SemaphoreType.DMA,
            pltpu.SemaphoreType.DMA,
        ],
    )
    def interp(tb_hbm, ab_hbm, out_hbm, tb_v, ab_v, o_v, sem0, sem1):
        wid = lax.axis_index("s") * _NC + lax.axis_index("c")
        base = wid * per_w
        cp0 = pltpu.async_copy(tb_hbm, tb_v, sem0)
        cp1 = pltpu.async_copy(ab_hbm.at[pl.ds(2 * base, 2 * per_w)], ab_v,
                               sem1)
        cp0.wait()
        cp1.wait()

        last = n_ext - 1
        for j in range(n_vec):
            t = base + j * _L + lax.iota(jnp.int32, _L)
            tf = t.astype(jnp.float32)
            pos = jnp.zeros((_L,), jnp.int32)
            for step in steps:
                cand = pos + step
                cand_c = jnp.minimum(cand, last) + n_pad
                pv = plsc.load_gather(tb_v, [cand_c])
                ok = (cand <= last) & (pv <= tf)
                pos = jnp.where(ok, cand, pos)
            left = jnp.minimum(pos, last - 1)
            y_b = plsc.load_gather(tb_v, [left])
            y_a = plsc.load_gather(tb_v, [left + 1])
            sl = pl.ds(j * _L, _L)
            o_v[sl] = ab_v[sl] * y_a + ab_v[pl.ds(per_w + j * _L, _L)] * y_b

        pltpu.sync_copy(o_v, out_hbm.at[pl.ds(base, per_w)])

    return interp


def kernel(LS_est, pilot_pos_1based, Nfft, interp_alpha, interp_beta):
    n_out = interp_alpha.shape[0]
    n_pil = LS_est.shape[0]
    per_w = n_out // _NW
    slope = (LS_est[-1] - LS_est[-2]) / (
        pilot_pos_1based[-1] - pilot_pos_1based[-2])
    h_ext = jnp.concatenate(
        [LS_est, LS_est[-1:] + slope * (Nfft - 1 - pilot_pos_1based[-1:])])
    p_last = jnp.reshape(Nfft - 1, (1,)).astype(pilot_pos_1based.dtype)
    p_ext = jnp.concatenate([pilot_pos_1based, p_last])
    n_ext = n_pil + 1
    pad = (-n_ext) % _L
    n_pad = n_ext + pad
    tb = jnp.concatenate([jnp.pad(h_ext, (0, pad)), jnp.pad(p_ext, (0, pad))])
    ab = jnp.stack(
        [interp_alpha.reshape(_NW, per_w), interp_beta.reshape(_NW, per_w)],
        axis=1).reshape(2 * n_out)
    return _build(n_ext, n_pad, n_out)(tb, ab)


# speedup vs baseline: 1.0537x; 1.0537x over previous
import functools

import jax
import jax.numpy as jnp
from jax import lax
from jax.experimental import pallas as pl
from jax.experimental.pallas import tpu as pltpu
from jax.experimental.pallas import tpu_sc as plsc

_NC = 2
_NS = 16
_NW = _NC * _NS
_L = 16


@functools.lru_cache(maxsize=None)
def _build(n_ext: int, n_pad: int, n_out: int):
    per_w = n_out // _NW
    n_vec = per_w // _L
    steps = []
    s = 1
    while s * 2 < n_ext:
        s *= 2
    while s >= 1:
        steps.append(s)
        s //= 2

    mesh = plsc.VectorSubcoreMesh(
        core_axis_name="c", subcore_axis_name="s",
        num_cores=_NC, num_subcores=_NS,
    )

    @functools.partial(
        pl.kernel,
        out_type=jax.ShapeDtypeStruct((n_out,), jnp.float32),
        mesh=mesh,
        compiler_params=pltpu.CompilerParams(needs_layout_passes=False),
        scratch_types=[
            pltpu.VMEM((2 * n_pad,), jnp.float32),
            pltpu.VMEM((2 * per_w,), jnp.float32),
            pltpu.VMEM((per_w,), jnp.float32),
            pltpu.SemaphoreType.DMA,
            pltpu.SemaphoreType.DMA,
        ],
    )
    def interp(tb_hbm, ab_hbm, out_hbm, tb_v, ab_v, o_v, sem0, sem1):
        wid = lax.axis_index("s") * _NC + lax.axis_index("c")
        base = wid * per_w
        cp0 = pltpu.async_copy(tb_hbm, tb_v, sem0)
        cp1 = pltpu.async_copy(ab_hbm.at[pl.ds(2 * base, 2 * per_w)], ab_v,
                               sem1)
        cp0.wait()
        cp1.wait()

        last = n_ext - 1
        for j in range(n_vec):
            t = base + j * _L + lax.iota(jnp.int32, _L)
            tf = t.astype(jnp.float32)
            pos = jnp.zeros((_L,), jnp.int32)
            for step in steps:
                cand = pos + step
                cand_c = jnp.minimum(cand, last) + n_pad
                pv = plsc.load_gather(tb_v, [cand_c])
                ok = (cand <= last) & (pv <= tf)
                pos = jnp.where(ok, cand, pos)
            left = jnp.minimum(pos, last - 1)
            y_b = plsc.load_gather(tb_v, [left])
            y_a = plsc.load_gather(tb_v, [left + 1])
            sl = pl.ds(j * _L, _L)
            o_v[sl] = ab_v[sl] * y_a + ab_v[pl.ds(per_w + j * _L, _L)] * y_b

        pltpu.sync_copy(o_v, out_hbm.at[pl.ds(base, per_w)])

    return interp


def kernel(LS_est, pilot_pos_1based, Nfft, interp_alpha, interp_beta):
    n_out = interp_alpha.shape[0]
    n_pil = LS_est.shape[0]
    per_w = n_out // _NW
    slope = (LS_est[-1] - LS_est[-2]) / (
        pilot_pos_1based[-1] - pilot_pos_1based[-2])
    h_ext = jnp.concatenate(
        [LS_est, LS_est[-1:] + slope * (Nfft - 1 - pilot_pos_1based[-1:])])
    p_last = jnp.reshape(Nfft - 1, (1,)).astype(pilot_pos_1based.dtype)
    p_ext = jnp.concatenate([pilot_pos_1based, p_last])
    n_ext = n_pil + 1
    pad = (-n_ext) % _L
    n_pad = n_ext + pad
    tb = jnp.concatenate([jnp.pad(h_ext, (0, pad)), jnp.pad(p_ext, (0, pad))])
    ab = jnp.stack(
        [interp_alpha.reshape(_NW, per_w), interp_beta.reshape(_NW, per_w)],
        axis=1).reshape(2 * n_out)
    return _build(n_ext, n_pad, n_out)(tb, ab)


# speedup vs baseline: 1.4298x; 1.3569x over previous
import functools

import jax
import jax.numpy as jnp
from jax import lax
from jax.experimental import pallas as pl
from jax.experimental.pallas import tpu as pltpu
from jax.experimental.pallas import tpu_sc as plsc

_NC = 2
_NS = 16
_NW = _NC * _NS
_L = 16


@functools.lru_cache(maxsize=None)
def _build(n_pil: int, n_out: int):
    per_w = n_out // _NW
    n_vec = per_w // _L
    steps = []
    s = 1
    while s * 2 < n_pil:
        s *= 2
    while s >= 1:
        steps.append(s)
        s //= 2

    mesh = plsc.VectorSubcoreMesh(
        core_axis_name="c", subcore_axis_name="s",
        num_cores=_NC, num_subcores=_NS,
    )

    @functools.partial(
        pl.kernel,
        out_type=jax.ShapeDtypeStruct((n_out,), jnp.float32),
        mesh=mesh,
        compiler_params=pltpu.CompilerParams(needs_layout_passes=False),
        scratch_types=[
            pltpu.VMEM((n_pil,), jnp.float32),
            pltpu.VMEM((n_pil,), jnp.float32),
            pltpu.VMEM((per_w,), jnp.float32),
            pltpu.VMEM((per_w,), jnp.float32),
            pltpu.VMEM((per_w,), jnp.float32),
            pltpu.SemaphoreType.DMA,
            pltpu.SemaphoreType.DMA,
        ],
    )
    def interp(h_hbm, p_hbm, a_hbm, b_hbm, out_hbm,
               h_v, p_v, a_v, b_v, o_v, sem0, sem1):
        wid = lax.axis_index("s") * _NC + lax.axis_index("c")
        base = wid * per_w
        cp_p = pltpu.async_copy(p_hbm, p_v, sem0)
        cp_h = pltpu.async_copy(h_hbm, h_v, sem0)
        cp_a = pltpu.async_copy(a_hbm.at[pl.ds(base, per_w)], a_v, sem1)
        cp_b = pltpu.async_copy(b_hbm.at[pl.ds(base, per_w)], b_v, sem1)
        cp_p.wait()
        cp_h.wait()
        cp_a.wait()
        cp_b.wait()

        last = n_pil - 1
        zero = jnp.zeros((_L,), jnp.int32)
        iota = lax.iota(jnp.int32, _L)
        tfs = [(base + j * _L + iota).astype(jnp.float32)
               for j in range(n_vec)]
        poss = [zero] * n_vec
        for step in steps:
            for j in range(n_vec):
                cand = poss[j] + step
                pv = plsc.load_gather(p_v, [jnp.minimum(cand, last)])
                ok = (cand <= last) & (pv <= tfs[j])
                poss[j] = jnp.where(ok, cand, poss[j])

        vlast = zero + last
        h_last = plsc.load_gather(h_v, [vlast])
        h_prev = plsc.load_gather(h_v, [vlast - 1])
        p_last = plsc.load_gather(p_v, [vlast])
        p_prev = plsc.load_gather(p_v, [vlast - 1])
        slope = (h_last - h_prev) / (p_last - p_prev)
        h_ext = h_last + slope * (float(n_out - 1) - p_last)

        for j in range(n_vec):
            left = poss[j]
            right = left + 1
            y_b = plsc.load_gather(h_v, [left])
            y_a = jnp.where(right > last, h_ext,
                            plsc.load_gather(h_v, [jnp.minimum(right, last)]))
            sl = pl.ds(j * _L, _L)
            o_v[sl] = a_v[sl] * y_a + b_v[sl] * y_b

        pltpu.sync_copy(o_v, out_hbm.at[pl.ds(base, per_w)])

    return interp


def kernel(LS_est, pilot_pos_1based, Nfft, interp_alpha, interp_beta):
    del Nfft
    n_out = interp_alpha.shape[0]
    n_pil = LS_est.shape[0]
    return _build(n_pil, n_out)(
        LS_est, pilot_pos_1based, interp_alpha, interp_beta)
